# Initial kernel scaffold; baseline (speedup 1.0000x reference)
import jax
import jax.numpy as jnp
from jax.experimental import pallas as pl


def kernel(x, edge_index, batch, W0, b0, g0, be0, W1, b1, g1, be1, W2, b2, g2, be2, Wp0, bp0, lg0, lb0, Wp1, bp1, lg1, lb1, Wp2, bp2):
    raise NotImplementedError("write your pallas kernel here")


# speedup vs baseline: 2.9484x; 2.9484x over previous
import functools

import jax
import jax.numpy as jnp
from jax import lax
from jax.experimental import pallas as pl
from jax.experimental.pallas import tpu as pltpu
from jax.experimental.pallas import tpu_sc as plsc

N = 10000
E = 320000
D = 128
H = 128
NB = 64
MZ = 1000

NCORES = 2
NSUB = 16
NW = NCORES * NSUB
CH = 128
EPWP = 10240
NCH = EPWP // CH
CPP = 40
EPAD = NW * EPWP
NP = 10240
SPT = NP // NSUB

_f32 = jnp.float32
_i32 = jnp.int32


def _sc_mesh():
    return plsc.VectorSubcoreMesh(core_axis_name="c", subcore_axis_name="s")


DCH = 128
DEPT = 20096
DNCH = DEPT // DCH
DW = 128


def _sc_deg(e_deg, ones_rows, z16):
    @functools.partial(
        pl.kernel,
        out_type=jax.ShapeDtypeStruct((NCORES, NP, DW), _f32),
        mesh=_sc_mesh(),
        scratch_types=[
            pltpu.VMEM((DNCH, DCH), _i32),
            pltpu.VMEM((DCH, DW), _f32),
            pltpu.VMEM_SHARED((NP, DW), _f32),
            pltpu.SemaphoreType.DMA,
        ],
    )
    def k(e_hbm, ones_hbm, z_hbm, out_hbm, idx_v, ones_v, acc, sem):
        cid = lax.axis_index("c")
        sid = lax.axis_index("s")

        pltpu.sync_copy(z_hbm.at[pl.ds(sid * SPT, SPT)],
                        acc.at[pl.ds(sid * SPT, SPT)])
        pltpu.sync_copy(e_hbm.at[cid, sid], idx_v)
        pltpu.sync_copy(ones_hbm, ones_v)
        plsc.subcore_barrier()

        def body(j, carry):
            @pl.when(j >= 4)
            def _():
                pltpu.make_async_copy(
                    ones_v, acc.at[idx_v.at[j]], sem).wait()
            pltpu.async_copy(ones_v, acc.at[idx_v.at[j]], sem, add=True)
            return carry

        lax.fori_loop(0, DNCH, body, 0)
        for _ in range(4):
            pltpu.make_async_copy(ones_v, acc.at[idx_v.at[0]], sem).wait()
        plsc.subcore_barrier()

        pltpu.sync_copy(acc.at[pl.ds(sid * SPT, SPT)],
                        out_hbm.at[cid, pl.ds(sid * SPT, SPT)])

    return k(e_deg, ones_rows, z16)


def _sc_agg(hs, srcs, dsts, z128):
    @functools.partial(
        pl.kernel,
        out_type=jax.ShapeDtypeStruct((NCORES, NP, D), _f32),
        mesh=_sc_mesh(),
        scratch_types=[
            pltpu.VMEM((CPP, CH), _i32),
            pltpu.VMEM((CPP, CH), _i32),
            pltpu.VMEM((2, CH, D), _f32),
            pltpu.VMEM_SHARED((NP, D), _f32),
            pltpu.SemaphoreType.DMA,
            pltpu.SemaphoreType.DMA,
            pltpu.SemaphoreType.DMA,
        ],
    )
    def k(hs_hbm, src_hbm, dst_hbm, z_hbm, out_hbm,
          src_v, dst_v, rows_v, acc, sem_g, sem_s0, sem_s1):
        cid = lax.axis_index("c")
        sid = lax.axis_index("s")
        wid = cid * NSUB + sid
        sems = (sem_s0, sem_s1)

        pltpu.sync_copy(z_hbm.at[pl.ds(sid * SPT, SPT)],
                        acc.at[pl.ds(sid * SPT, SPT)])
        plsc.subcore_barrier()

        def chunk_step(c, b, drain):
            if drain:
                pltpu.make_async_copy(
                    rows_v.at[b], acc.at[dst_v.at[c]], sems[b]).wait()
            pltpu.async_copy(
                hs_hbm.at[src_v.at[c]], rows_v.at[b], sem_g).wait()
            pltpu.async_copy(
                rows_v.at[b], acc.at[dst_v.at[c]], sems[b], add=True)

        for p in range(2):
            pltpu.sync_copy(src_hbm.at[wid, pl.ds(p * CPP, CPP)], src_v)
            pltpu.sync_copy(dst_hbm.at[wid, pl.ds(p * CPP, CPP)], dst_v)

            for b in range(2):
                chunk_step(b, b, drain=False)

            def body(j, carry):
                for b in range(2):
                    chunk_step(j * 2 + b, b, drain=True)
                return carry

            lax.fori_loop(1, CPP // 2, body, 0)

            for b in range(2):
                pltpu.make_async_copy(
                    rows_v.at[b], acc.at[dst_v.at[b]], sems[b]).wait()

        plsc.subcore_barrier()
        pltpu.sync_copy(acc.at[pl.ds(sid * SPT, SPT)],
                        out_hbm.at[cid, pl.ds(sid * SPT, SPT)])

    return k(hs, srcs, dsts, z128)


def _tc_prep(x, degT):
    def body(x_ref, d_ref, hs_ref, nn_ref):
        d = d_ref[...]
        deg_o = d[:, 0:1]
        deg_i = d[:, 1:2]
        ns = lax.rsqrt(jnp.maximum(deg_o, 1.0))
        nd = lax.rsqrt(jnp.maximum(deg_i, 1.0))
        hs_ref[...] = x_ref[...] * ns
        nn_ref[...] = jnp.concatenate([ns, nd], axis=1)

    return pl.pallas_call(
        body,
        out_shape=(jax.ShapeDtypeStruct((N, D), _f32),
                   jax.ShapeDtypeStruct((N, 2), _f32)),
    )(x, degT)


def _bn_block(a, g, be):
    mu = jnp.mean(a, axis=0, keepdims=True)
    var = jnp.mean((a - mu) ** 2, axis=0, keepdims=True)
    return (a - mu) * lax.rsqrt(var + 1e-5) * g + be


def _tc_layer(agg2, nn, W, b, g, be):
    def body(a_ref, nn_ref, W_ref, b_ref, g_ref, be_ref, o_ref):
        nnv = nn_ref[...]
        agg = (a_ref[0] + a_ref[1]) * nnv[:, 1:2]
        z = jnp.dot(agg, W_ref[...], preferred_element_type=_f32) + b_ref[...]
        h = _bn_block(jnp.maximum(z, 0.0), g_ref[...], be_ref[...])
        o_ref[...] = h * nnv[:, 0:1]

    return pl.pallas_call(
        body,
        out_shape=jax.ShapeDtypeStruct((N, D), _f32),
    )(agg2, nn, W, b, g, be)


def _ln_rows(z, g, b):
    mu = jnp.mean(z, axis=-1, keepdims=True)
    var = jnp.mean((z - mu) ** 2, axis=-1, keepdims=True)
    return (z - mu) * lax.rsqrt(var + 1e-5) * g + b


def _tc_final(agg2, nn, W2, b2, g2, be2, batchT,
              Wp0, bp0, lg0, lb0, Wp1, bp1, lg1, lb1, Wp2, bp2):
    def body(a_ref, nn_ref, W_ref, b_ref, g_ref, be_ref, bt_ref,
             Wp0_ref, bp0_ref, lg0_ref, lb0_ref,
             Wp1_ref, bp1_ref, lg1_ref, lb1_ref,
             Wp2_ref, bp2_ref, o_ref):
        nnv = nn_ref[...]
        agg = (a_ref[0] + a_ref[1]) * nnv[:, 1:2]
        z = jnp.dot(agg, W_ref[...], preferred_element_type=_f32) + b_ref[...]
        h = _bn_block(jnp.maximum(z, 0.0), g_ref[...], be_ref[...])

        oh = (bt_ref[...] == lax.broadcasted_iota(_i32, (NB, N), 0))
        hg = jnp.dot(oh.astype(_f32), h, preferred_element_type=_f32)

        z0 = jnp.dot(hg, Wp0_ref[...], preferred_element_type=_f32) + bp0_ref[...]
        z0 = jnp.maximum(_ln_rows(z0, lg0_ref[...], lb0_ref[...]), 0.0)
        z1 = jnp.dot(z0, Wp1_ref[...], preferred_element_type=_f32) + bp1_ref[...]
        z1 = jnp.maximum(_ln_rows(z1, lg1_ref[...], lb1_ref[...]), 0.0)
        z2 = jnp.dot(z1, Wp2_ref[...], preferred_element_type=_f32) + bp2_ref[...]
        o_ref[...] = jax.nn.sigmoid(z2)

    return pl.pallas_call(
        body,
        out_shape=jax.ShapeDtypeStruct((NB, MZ), _f32),
    )(agg2, nn, W2, b2, g2, be2, batchT,
      Wp0, bp0, lg0, lb0, Wp1, bp1, lg1, lb1, Wp2, bp2)


def kernel(x, edge_index, batch, W0, b0, g0, be0, W1, b1, g1, be1,
           W2, b2, g2, be2, Wp0, bp0, lg0, lb0, Wp1, bp1, lg1, lb1, Wp2, bp2):
    ei = edge_index.astype(_i32)
    dpad = jnp.full((NSUB * DEPT - E,), N, _i32)
    e_deg = jnp.stack(
        [jnp.concatenate([ei[0], dpad]), jnp.concatenate([ei[1], dpad])]
    ).reshape(2, NSUB, DNCH, DCH)
    pad_n = EPAD - E
    src_a = jnp.concatenate(
        [ei[0], jnp.zeros((pad_n,), _i32)]).reshape(NW, NCH, CH)
    dst_a = jnp.concatenate(
        [ei[1], jnp.full((pad_n,), N, _i32)]).reshape(NW, NCH, CH)
    z16 = jnp.zeros((NP, DW), _f32)
    z128 = jnp.zeros((NP, D), _f32)
    ones_rows = jnp.ones((DCH, DW), _f32)
    batchT = batch.astype(_i32).reshape(1, N)

    degs = _sc_deg(e_deg, ones_rows, z16)
    degT = degs[:, :N, 0].T
    hs, nn = _tc_prep(x, degT)

    params = ((W0, b0, g0, be0), (W1, b1, g1, be1))
    for (W, b, g, be) in params:
        agg2 = _sc_agg(hs, src_a, dst_a, z128)[:, :N]
        hs = _tc_layer(agg2, nn, W, b.reshape(1, H), g.reshape(1, H),
                       be.reshape(1, H))

    agg2 = _sc_agg(hs, src_a, dst_a, z128)[:, :N]
    return _tc_final(
        agg2, nn, W2, b2.reshape(1, H), g2.reshape(1, H), be2.reshape(1, H),
        batchT,
        Wp0, bp0.reshape(1, 2 * H), lg0.reshape(1, 2 * H), lb0.reshape(1, 2 * H),
        Wp1, bp1.reshape(1, H), lg1.reshape(1, H), lb1.reshape(1, H),
        Wp2, bp2.reshape(1, MZ))
